# baseline (speedup 1.0000x reference)
import functools

import jax
import jax.numpy as jnp
from jax import lax
from jax.experimental import pallas as pl
from jax.experimental.pallas import tpu as pltpu
from jax.experimental.pallas import tpu_sc as plsc

N = 10000
E = 320000
D = 128
HID = 128
R_HID = 256
R_OUT = 32

NC = 2
NS = 16
NW = NC * NS
EPW = E // NW
CK = 100
NCHUNK = EPW // CK
TF = 640
TL = 400
WBF = 128
WBL = 80
NWB = 5

_EPS = 1e-5


def _leaky(x):
    return jnp.where(x >= 0, x, 0.01 * x)


def _sc_mesh():
    return plsc.VectorSubcoreMesh(
        core_axis_name="c", subcore_axis_name="s", num_cores=NC, num_subcores=NS
    )


_SC_PARAMS = pltpu.CompilerParams(use_tc_tiling_on_sc=False,
                                  needs_layout_passes=False)


def _deg_body(src_hbm, dst_hbm, cs_hbm, cd_hbm,
              idx_s, idx_d, ones_v, tmp16, sem, acc_s, acc_d):
    c = lax.axis_index("c")
    s = lax.axis_index("s")
    wid = c * NS + s
    row0 = s * TF
    last = s == NS - 1

    pltpu.sync_copy(src_hbm.at[wid], idx_s)
    pltpu.sync_copy(dst_hbm.at[wid], idx_d)

    def _fill(i, _):
        ones_v[i, :] = jnp.ones((16,), jnp.float32)
        return 0
    lax.fori_loop(0, CK, _fill, 0)

    def _zero(i, _):
        tmp16[i, :] = jnp.zeros((16,), jnp.float32)
        return 0
    lax.fori_loop(0, TF, _zero, 0)

    @pl.when(jnp.logical_not(last))
    def _():
        pltpu.sync_copy(tmp16, acc_s.at[pl.ds(row0, TF)])
        pltpu.sync_copy(tmp16, acc_d.at[pl.ds(row0, TF)])

    @pl.when(last)
    def _():
        pltpu.sync_copy(tmp16.at[pl.ds(0, TL)], acc_s.at[pl.ds(row0, TL)])
        pltpu.sync_copy(tmp16.at[pl.ds(0, TL)], acc_d.at[pl.ds(row0, TL)])

    plsc.subcore_barrier()

    def _chunk(g, _):
        h1 = pltpu.async_copy(ones_v, acc_s.at[idx_s.at[g]], sem, add=True)
        h2 = pltpu.async_copy(ones_v, acc_d.at[idx_d.at[g]], sem, add=True)
        h1.wait()
        h2.wait()
        return 0
    lax.fori_loop(0, NCHUNK, _chunk, 0)
    plsc.subcore_barrier()

    def _write(acc, out_hbm, nrows):
        pltpu.sync_copy(acc.at[pl.ds(row0, nrows)],
                        out_hbm.at[c, pl.ds(row0, nrows)])

    @pl.when(jnp.logical_not(last))
    def _():
        _write(acc_s, cs_hbm, TF)
        _write(acc_d, cd_hbm, TF)

    @pl.when(last)
    def _():
        _write(acc_s, cs_hbm, TL)
        _write(acc_d, cd_hbm, TL)


def _deg_call(src3, dst3):
    return pl.kernel(
        _deg_body,
        out_type=(
            jax.ShapeDtypeStruct((NC, N, 16), jnp.float32),
            jax.ShapeDtypeStruct((NC, N, 16), jnp.float32),
        ),
        mesh=_sc_mesh(),
        compiler_params=_SC_PARAMS,
        scratch_types=(
            pltpu.VMEM((NCHUNK, CK), jnp.int32),
            pltpu.VMEM((NCHUNK, CK), jnp.int32),
            pltpu.VMEM((CK, 16), jnp.float32),
            pltpu.VMEM((TF, 16), jnp.float32),
            pltpu.SemaphoreType.DMA,
            pltpu.VMEM_SHARED((N, 16), jnp.float32),
            pltpu.VMEM_SHARED((N, 16), jnp.float32),
        ),
    )(src3, dst3)


NSEG = 5
QS = NCHUNK // NSEG
ESEG = EPW // NSEG
ZWB = 80


def _agg_body(y_hbm, src_hbm, dst_hbm, ew_hbm, p_hbm,
              sidx, didx, ewh, rows0, rows1, rows2,
              gs0, gs1, gs2, ss0, ss1, ss2, acc):
    c = lax.axis_index("c")
    s = lax.axis_index("s")
    wid = c * NS + s
    row0 = s * TF
    last = s == NS - 1
    nzwb = TF // ZWB

    def _zrow(i, _):
        for j in range(8):
            rows0[i, pl.ds(j * 16, 16)] = jnp.zeros((16,), jnp.float32)
        return 0
    lax.fori_loop(0, ZWB, _zrow, 0)

    @pl.when(jnp.logical_not(last))
    def _():
        for k in range(nzwb):
            pltpu.sync_copy(rows0.at[pl.ds(0, ZWB)],
                            acc.at[pl.ds(row0 + k * ZWB, ZWB)])

    @pl.when(last)
    def _():
        for k in range(TL // ZWB):
            pltpu.sync_copy(rows0.at[pl.ds(0, ZWB)],
                            acc.at[pl.ds(row0 + k * ZWB, ZWB)])

    plsc.subcore_barrier()

    def _scale(rows, g):
        def _one(t, _):
            k0 = 2 * t
            k1 = k0 + 1
            w0 = plsc.load_gather(
                ewh, [jnp.full((16,), g * CK + k0, jnp.int32)])
            w1 = plsc.load_gather(
                ewh, [jnp.full((16,), g * CK + k1, jnp.int32)])
            for j in range(8):
                rows[k0, pl.ds(j * 16, 16)] = rows[k0, pl.ds(j * 16, 16)] * w0
            for j in range(8):
                rows[k1, pl.ds(j * 16, 16)] = rows[k1, pl.ds(j * 16, 16)] * w1
            return 0
        lax.fori_loop(0, CK // 2, _one, 0)

    bufs = ((rows0, gs0, ss0), (rows1, gs1, ss1), (rows2, gs2, ss2))

    for q in range(NSEG):
        pltpu.sync_copy(src_hbm.at[wid, pl.ds(q * QS, QS)], sidx)
        pltpu.sync_copy(dst_hbm.at[wid, pl.ds(q * QS, QS)], didx)
        pltpu.sync_copy(ew_hbm.at[pl.ds(wid * EPW + q * ESEG, ESEG)], ewh)

        pltpu.async_copy(y_hbm.at[sidx.at[0]], rows0, gs0)
        pltpu.async_copy(y_hbm.at[sidx.at[1]], rows1, gs1)

        def _chunk(g, _):
            b = lax.rem(g, 3)
            for i, (buf, gsem, ssem) in enumerate(bufs):
                @pl.when(b == i)
                def _(buf=buf, gsem=gsem, ssem=ssem):
                    pltpu.make_async_copy(
                        y_hbm.at[sidx.at[g]], buf, gsem).wait()
                    _scale(buf, g)
                    pltpu.async_copy(buf, acc.at[didx.at[g]], ssem, add=True)

            gn = g + 2
            bn = lax.rem(gn, 3)

            @pl.when(gn < QS)
            def _():
                for i, (buf, gsem, ssem) in enumerate(bufs):
                    @pl.when(bn == i)
                    def _(buf=buf, gsem=gsem, ssem=ssem):
                        @pl.when(g >= 1)
                        def _():
                            pltpu.make_async_copy(
                                buf, acc.at[didx.at[g - 1]], ssem).wait()
                        pltpu.async_copy(y_hbm.at[sidx.at[gn]], buf, gsem)

            return 0
        lax.fori_loop(0, QS, _chunk, 0)

        for g in (QS - 3, QS - 2, QS - 1):
            buf, _, ssem = bufs[g % 3]
            pltpu.make_async_copy(buf, acc.at[didx.at[g]], ssem).wait()
    plsc.subcore_barrier()

    @pl.when(jnp.logical_not(last))
    def _():
        for k in range(nzwb):
            pltpu.sync_copy(acc.at[pl.ds(row0 + k * ZWB, ZWB)],
                            p_hbm.at[c, pl.ds(row0 + k * ZWB, ZWB)])

    @pl.when(last)
    def _():
        for k in range(TL // ZWB):
            pltpu.sync_copy(acc.at[pl.ds(row0 + k * ZWB, ZWB)],
                            p_hbm.at[c, pl.ds(row0 + k * ZWB, ZWB)])


def _agg_call(y, src3, dst3, ew):
    return pl.kernel(
        _agg_body,
        out_type=jax.ShapeDtypeStruct((NC, N, D), jnp.float32),
        mesh=_sc_mesh(),
        compiler_params=_SC_PARAMS,
        scratch_types=(
            pltpu.VMEM((QS, CK), jnp.int32),
            pltpu.VMEM((QS, CK), jnp.int32),
            pltpu.VMEM((ESEG,), jnp.float32),
            pltpu.VMEM((CK, D), jnp.float32),
            pltpu.VMEM((CK, D), jnp.float32),
            pltpu.VMEM((CK, D), jnp.float32),
            pltpu.SemaphoreType.DMA,
            pltpu.SemaphoreType.DMA,
            pltpu.SemaphoreType.DMA,
            pltpu.SemaphoreType.DMA,
            pltpu.SemaphoreType.DMA,
            pltpu.SemaphoreType.DMA,
            pltpu.VMEM_SHARED((N, D), jnp.float32),
        ),
    )(y, src3, dst3, ew)


BN = 1000
NB = N // BN

_DOT = functools.partial(jnp.dot, preferred_element_type=jnp.float32,
                         precision=lax.Precision.HIGHEST)


def _dinv(cnt_block):
    deg = jnp.maximum(cnt_block[0, :, 0:1] + cnt_block[1, :, 0:1], 1.0)
    return lax.rsqrt(deg)


def _y1_body(cs_ref, x_ref, w_ref, y_ref):
    h = x_ref[...] * _dinv(cs_ref[...])
    y_ref[...] = _DOT(h, w_ref[...])


def _y1_call(cnt_s, x, w1):
    return pl.pallas_call(
        _y1_body,
        grid=(NB,),
        in_specs=[
            pl.BlockSpec((NC, BN, 16), lambda i: (0, i, 0)),
            pl.BlockSpec((BN, D), lambda i: (i, 0)),
            pl.BlockSpec((D, HID), lambda i: (0, 0)),
        ],
        out_specs=pl.BlockSpec((BN, HID), lambda i: (i, 0)),
        out_shape=jax.ShapeDtypeStruct((N, HID), jnp.float32),
    )(cnt_s, x, w1)


def _gnorm_h(x, s1_ref, s2_ref, al_ref, ga_ref, be_ref):
    mu = s1_ref[...] * (1.0 / N)
    ex2 = s2_ref[...] * (1.0 / N)
    al = al_ref[...]
    var = ex2 - (2.0 * al - al * al) * mu * mu
    sub = x - al * mu
    return _leaky(ga_ref[...] * sub * lax.rsqrt(var + _EPS) + be_ref[...])


def _layer1_body(p_ref, cs_ref, cd_ref, al_ref, ga_ref, be_ref,
                 pw_ref, pb_ref, w2_ref, y2_ref, ps_ref, s1_ref, s2_ref):
    ph = pl.program_id(0)
    i = pl.program_id(1)
    x = (p_ref[0] + p_ref[1]) * _dinv(cd_ref[...])

    @pl.when(ph == 0)
    def _():
        @pl.when(i == 0)
        def _():
            s1_ref[...] = jnp.zeros_like(s1_ref)
            s2_ref[...] = jnp.zeros_like(s2_ref)
        s1_ref[...] += jnp.sum(x, axis=0, keepdims=True)
        s2_ref[...] += jnp.sum(x * x, axis=0, keepdims=True)

    @pl.when(ph == 1)
    def _():
        h = _gnorm_h(x, s1_ref, s2_ref, al_ref, ga_ref, be_ref)
        phis = _leaky(_DOT(h, pw_ref[...]) + pb_ref[...])

        @pl.when(i == 0)
        def _():
            ps_ref[...] = jnp.zeros_like(ps_ref)
        ps_ref[...] += jnp.sum(phis, axis=0, keepdims=True)
        y2_ref[...] = _DOT(h * _dinv(cs_ref[...]), w2_ref[...])


def _layer1_call(p, cnt_s, cnt_d, al, ga, be, pw, pb, w2):
    return pl.pallas_call(
        _layer1_body,
        grid=(2, NB),
        in_specs=[
            pl.BlockSpec((NC, BN, HID), lambda p_, i: (0, i, 0)),
            pl.BlockSpec((NC, BN, 16), lambda p_, i: (0, i, 0)),
            pl.BlockSpec((NC, BN, 16), lambda p_, i: (0, i, 0)),
            pl.BlockSpec((1, HID), lambda p_, i: (0, 0)),
            pl.BlockSpec((1, HID), lambda p_, i: (0, 0)),
            pl.BlockSpec((1, HID), lambda p_, i: (0, 0)),
            pl.BlockSpec((HID, R_HID), lambda p_, i: (0, 0)),
            pl.BlockSpec((1, R_HID), lambda p_, i: (0, 0)),
            pl.BlockSpec((HID, HID), lambda p_, i: (0, 0)),
        ],
        out_specs=[
            pl.BlockSpec((BN, HID), lambda p_, i: (i, 0)),
            pl.BlockSpec((1, R_HID), lambda p_, i: (0, 0)),
        ],
        out_shape=[
            jax.ShapeDtypeStruct((N, HID), jnp.float32),
            jax.ShapeDtypeStruct((1, R_HID), jnp.float32),
        ],
        scratch_shapes=[
            pltpu.VMEM((1, HID), jnp.float32),
            pltpu.VMEM((1, HID), jnp.float32),
        ],
    )(p, cnt_s, cnt_d, al, ga, be, pw, pb, w2)


def _layer2_body(p_ref, cd_ref, al_ref, ga_ref, be_ref, pw_ref, pb_ref,
                 ps1_ref, rw1_ref, rb1_ref, rw2_ref, rb2_ref,
                 o_ref, s1_ref, s2_ref, ps_ref):
    ph = pl.program_id(0)
    i = pl.program_id(1)
    x = (p_ref[0] + p_ref[1]) * _dinv(cd_ref[...])

    @pl.when(ph == 0)
    def _():
        @pl.when(i == 0)
        def _():
            s1_ref[...] = jnp.zeros_like(s1_ref)
            s2_ref[...] = jnp.zeros_like(s2_ref)
        s1_ref[...] += jnp.sum(x, axis=0, keepdims=True)
        s2_ref[...] += jnp.sum(x * x, axis=0, keepdims=True)

    @pl.when(ph == 1)
    def _():
        h = _gnorm_h(x, s1_ref, s2_ref, al_ref, ga_ref, be_ref)
        phis = _leaky(_DOT(h, pw_ref[...]) + pb_ref[...])

        @pl.when(i == 0)
        def _():
            ps_ref[...] = jnp.zeros_like(ps_ref)
        ps_ref[...] += jnp.sum(phis, axis=0, keepdims=True)

        @pl.when(i == NB - 1)
        def _():
            r1 = _leaky(_DOT(ps1_ref[...], rw1_ref[...]) + rb1_ref[...])
            r2 = _leaky(_DOT(ps_ref[...], rw2_ref[...]) + rb2_ref[...])
            o_ref[...] = _leaky(jnp.concatenate([r1, r2], axis=1))


def _layer2_call(p, cnt_d, al, ga, be, pw, pb, ps1, rw1, rb1, rw2, rb2):
    return pl.pallas_call(
        _layer2_body,
        grid=(2, NB),
        in_specs=[
            pl.BlockSpec((NC, BN, HID), lambda p_, i: (0, i, 0)),
            pl.BlockSpec((NC, BN, 16), lambda p_, i: (0, i, 0)),
            pl.BlockSpec((1, HID), lambda p_, i: (0, 0)),
            pl.BlockSpec((1, HID), lambda p_, i: (0, 0)),
            pl.BlockSpec((1, HID), lambda p_, i: (0, 0)),
            pl.BlockSpec((HID, R_HID), lambda p_, i: (0, 0)),
            pl.BlockSpec((1, R_HID), lambda p_, i: (0, 0)),
            pl.BlockSpec((1, R_HID), lambda p_, i: (0, 0)),
            pl.BlockSpec((R_HID, R_OUT), lambda p_, i: (0, 0)),
            pl.BlockSpec((1, R_OUT), lambda p_, i: (0, 0)),
            pl.BlockSpec((R_HID, R_OUT), lambda p_, i: (0, 0)),
            pl.BlockSpec((1, R_OUT), lambda p_, i: (0, 0)),
        ],
        out_specs=pl.BlockSpec((1, 2 * R_OUT), lambda p_, i: (0, 0)),
        out_shape=jax.ShapeDtypeStruct((1, 2 * R_OUT), jnp.float32),
        scratch_shapes=[
            pltpu.VMEM((1, HID), jnp.float32),
            pltpu.VMEM((1, HID), jnp.float32),
            pltpu.VMEM((1, R_HID), jnp.float32),
        ],
    )(p, cnt_d, al, ga, be, pw, pb, ps1, rw1, rb1, rw2, rb2)


def kernel(features, edge_index, edge_weights, W1, W2,
           gn1_alpha, gn1_gamma, gn1_beta, gn2_alpha, gn2_gamma, gn2_beta,
           r1_phi_W, r1_phi_b, r1_rho_W, r1_rho_b,
           r2_phi_W, r2_phi_b, r2_rho_W, r2_rho_b):
    src3 = edge_index[0].reshape(NW, NCHUNK, CK)
    dst3 = edge_index[1].reshape(NW, NCHUNK, CK)

    cnt_s, cnt_d = _deg_call(src3, dst3)

    row = lambda v: v.reshape(1, -1)

    y1 = _y1_call(cnt_s, features, W1)
    p1 = _agg_call(y1, src3, dst3, edge_weights)
    y2, ps1 = _layer1_call(p1, cnt_s, cnt_d,
                           row(gn1_alpha), row(gn1_gamma), row(gn1_beta),
                           r1_phi_W, row(r1_phi_b), W2)
    p2 = _agg_call(y2, src3, dst3, edge_weights)
    return _layer2_call(p2, cnt_d,
                        row(gn2_alpha), row(gn2_gamma), row(gn2_beta),
                        r2_phi_W, row(r2_phi_b),
                        ps1, r1_rho_W, row(r1_rho_b),
                        r2_rho_W, row(r2_rho_b))


# speedup vs baseline: 1.0370x; 1.0370x over previous
import functools

import jax
import jax.numpy as jnp
from jax import lax
from jax.experimental import pallas as pl
from jax.experimental.pallas import tpu as pltpu
from jax.experimental.pallas import tpu_sc as plsc

N = 10000
E = 320000
D = 128
HID = 128
R_HID = 256
R_OUT = 32

NC = 2
NS = 16
NW = NC * NS
EPW = E // NW
CK = 80
NCHUNK = EPW // CK
TF = 640
TL = 400
WBF = 128
WBL = 80
NWB = 5

_EPS = 1e-5


def _leaky(x):
    return jnp.where(x >= 0, x, 0.01 * x)


def _sc_mesh():
    return plsc.VectorSubcoreMesh(
        core_axis_name="c", subcore_axis_name="s", num_cores=NC, num_subcores=NS
    )


_SC_PARAMS = pltpu.CompilerParams(use_tc_tiling_on_sc=False,
                                  needs_layout_passes=False)


def _deg_body(src_hbm, dst_hbm, cs_hbm, cd_hbm,
              idx_s, idx_d, ones_v, tmp16, sem, acc_s, acc_d):
    c = lax.axis_index("c")
    s = lax.axis_index("s")
    wid = c * NS + s
    row0 = s * TF
    last = s == NS - 1

    pltpu.sync_copy(src_hbm.at[wid], idx_s)
    pltpu.sync_copy(dst_hbm.at[wid], idx_d)

    def _fill(i, _):
        ones_v[i, :] = jnp.ones((16,), jnp.float32)
        return 0
    lax.fori_loop(0, CK, _fill, 0)

    def _zero(i, _):
        tmp16[i, :] = jnp.zeros((16,), jnp.float32)
        return 0
    lax.fori_loop(0, TF, _zero, 0)

    @pl.when(jnp.logical_not(last))
    def _():
        pltpu.sync_copy(tmp16, acc_s.at[pl.ds(row0, TF)])
        pltpu.sync_copy(tmp16, acc_d.at[pl.ds(row0, TF)])

    @pl.when(last)
    def _():
        pltpu.sync_copy(tmp16.at[pl.ds(0, TL)], acc_s.at[pl.ds(row0, TL)])
        pltpu.sync_copy(tmp16.at[pl.ds(0, TL)], acc_d.at[pl.ds(row0, TL)])

    plsc.subcore_barrier()

    def _chunk(g, _):
        h1 = pltpu.async_copy(ones_v, acc_s.at[idx_s.at[g]], sem, add=True)
        h2 = pltpu.async_copy(ones_v, acc_d.at[idx_d.at[g]], sem, add=True)
        h1.wait()
        h2.wait()
        return 0
    lax.fori_loop(0, NCHUNK, _chunk, 0)
    plsc.subcore_barrier()

    def _write(acc, out_hbm, nrows):
        pltpu.sync_copy(acc.at[pl.ds(row0, nrows)],
                        out_hbm.at[c, pl.ds(row0, nrows)])

    @pl.when(jnp.logical_not(last))
    def _():
        _write(acc_s, cs_hbm, TF)
        _write(acc_d, cd_hbm, TF)

    @pl.when(last)
    def _():
        _write(acc_s, cs_hbm, TL)
        _write(acc_d, cd_hbm, TL)


def _deg_call(src3, dst3):
    return pl.kernel(
        _deg_body,
        out_type=(
            jax.ShapeDtypeStruct((NC, N, 16), jnp.float32),
            jax.ShapeDtypeStruct((NC, N, 16), jnp.float32),
        ),
        mesh=_sc_mesh(),
        compiler_params=_SC_PARAMS,
        scratch_types=(
            pltpu.VMEM((NCHUNK, CK), jnp.int32),
            pltpu.VMEM((NCHUNK, CK), jnp.int32),
            pltpu.VMEM((CK, 16), jnp.float32),
            pltpu.VMEM((TF, 16), jnp.float32),
            pltpu.SemaphoreType.DMA,
            pltpu.VMEM_SHARED((N, 16), jnp.float32),
            pltpu.VMEM_SHARED((N, 16), jnp.float32),
        ),
    )(src3, dst3)


NSEG = 5
QS = NCHUNK // NSEG
ESEG = EPW // NSEG
ZWB = 80


def _agg_body(y_hbm, src_hbm, dst_hbm, ew_hbm, p_hbm,
              sidx, didx, ewh, rows0, rows1, rows2, rows3,
              gs0, gs1, gs2, gs3, ss0, ss1, ss2, ss3, acc):
    c = lax.axis_index("c")
    s = lax.axis_index("s")
    wid = c * NS + s
    row0 = s * TF
    last = s == NS - 1
    nzwb = TF // ZWB

    def _zrow(i, _):
        for j in range(8):
            rows0[i, pl.ds(j * 16, 16)] = jnp.zeros((16,), jnp.float32)
        return 0
    lax.fori_loop(0, ZWB, _zrow, 0)

    @pl.when(jnp.logical_not(last))
    def _():
        for k in range(nzwb):
            pltpu.sync_copy(rows0.at[pl.ds(0, ZWB)],
                            acc.at[pl.ds(row0 + k * ZWB, ZWB)])

    @pl.when(last)
    def _():
        for k in range(TL // ZWB):
            pltpu.sync_copy(rows0.at[pl.ds(0, ZWB)],
                            acc.at[pl.ds(row0 + k * ZWB, ZWB)])

    plsc.subcore_barrier()

    def _scale(rows, g):
        def _one(t, _):
            k0 = 2 * t
            k1 = k0 + 1
            w0 = plsc.load_gather(
                ewh, [jnp.full((16,), g * CK + k0, jnp.int32)])
            w1 = plsc.load_gather(
                ewh, [jnp.full((16,), g * CK + k1, jnp.int32)])
            for j in range(8):
                rows[k0, pl.ds(j * 16, 16)] = rows[k0, pl.ds(j * 16, 16)] * w0
            for j in range(8):
                rows[k1, pl.ds(j * 16, 16)] = rows[k1, pl.ds(j * 16, 16)] * w1
            return 0
        lax.fori_loop(0, CK // 2, _one, 0)

    bufs = ((rows0, gs0, ss0), (rows1, gs1, ss1),
            (rows2, gs2, ss2), (rows3, gs3, ss3))
    NBUF = len(bufs)
    PF = NBUF - 1

    for q in range(NSEG):
        pltpu.sync_copy(src_hbm.at[wid, pl.ds(q * QS, QS)], sidx)
        pltpu.sync_copy(dst_hbm.at[wid, pl.ds(q * QS, QS)], didx)
        pltpu.sync_copy(ew_hbm.at[pl.ds(wid * EPW + q * ESEG, ESEG)], ewh)

        for g0 in range(PF):
            pltpu.async_copy(y_hbm.at[sidx.at[g0]], bufs[g0][0], bufs[g0][1])

        def _chunk(g, _):
            b = lax.rem(g, NBUF)
            for i, (buf, gsem, ssem) in enumerate(bufs):
                @pl.when(b == i)
                def _(buf=buf, gsem=gsem, ssem=ssem):
                    pltpu.make_async_copy(
                        y_hbm.at[sidx.at[g]], buf, gsem).wait()
                    _scale(buf, g)
                    pltpu.async_copy(buf, acc.at[didx.at[g]], ssem, add=True)

            gn = g + PF
            bn = lax.rem(gn, NBUF)

            @pl.when(gn < QS)
            def _():
                for i, (buf, gsem, ssem) in enumerate(bufs):
                    @pl.when(bn == i)
                    def _(buf=buf, gsem=gsem, ssem=ssem):
                        @pl.when(g >= 1)
                        def _():
                            pltpu.make_async_copy(
                                buf, acc.at[didx.at[g - 1]], ssem).wait()
                        pltpu.async_copy(y_hbm.at[sidx.at[gn]], buf, gsem)

            return 0
        lax.fori_loop(0, QS, _chunk, 0)

        for g in range(QS - NBUF, QS):
            buf, _, ssem = bufs[g % NBUF]
            pltpu.make_async_copy(buf, acc.at[didx.at[g]], ssem).wait()
    plsc.subcore_barrier()

    @pl.when(jnp.logical_not(last))
    def _():
        for k in range(nzwb):
            pltpu.sync_copy(acc.at[pl.ds(row0 + k * ZWB, ZWB)],
                            p_hbm.at[c, pl.ds(row0 + k * ZWB, ZWB)])

    @pl.when(last)
    def _():
        for k in range(TL // ZWB):
            pltpu.sync_copy(acc.at[pl.ds(row0 + k * ZWB, ZWB)],
                            p_hbm.at[c, pl.ds(row0 + k * ZWB, ZWB)])


def _agg_call(y, src3, dst3, ew):
    return pl.kernel(
        _agg_body,
        out_type=jax.ShapeDtypeStruct((NC, N, D), jnp.float32),
        mesh=_sc_mesh(),
        compiler_params=_SC_PARAMS,
        scratch_types=(
            pltpu.VMEM((QS, CK), jnp.int32),
            pltpu.VMEM((QS, CK), jnp.int32),
            pltpu.VMEM((ESEG,), jnp.float32),
            pltpu.VMEM((CK, D), jnp.float32),
            pltpu.VMEM((CK, D), jnp.float32),
            pltpu.VMEM((CK, D), jnp.float32),
            pltpu.VMEM((CK, D), jnp.float32),
            pltpu.SemaphoreType.DMA,
            pltpu.SemaphoreType.DMA,
            pltpu.SemaphoreType.DMA,
            pltpu.SemaphoreType.DMA,
            pltpu.SemaphoreType.DMA,
            pltpu.SemaphoreType.DMA,
            pltpu.SemaphoreType.DMA,
            pltpu.SemaphoreType.DMA,
            pltpu.VMEM_SHARED((N, D), jnp.float32),
        ),
    )(y, src3, dst3, ew)


BN = 1000
NB = N // BN

_DOT = functools.partial(jnp.dot, preferred_element_type=jnp.float32,
                         precision=lax.Precision.HIGHEST)


def _dinv(cnt_block):
    deg = jnp.maximum(cnt_block[0, :, 0:1] + cnt_block[1, :, 0:1], 1.0)
    return lax.rsqrt(deg)


def _y1_body(cs_ref, x_ref, w_ref, y_ref):
    h = x_ref[...] * _dinv(cs_ref[...])
    y_ref[...] = _DOT(h, w_ref[...])


def _y1_call(cnt_s, x, w1):
    return pl.pallas_call(
        _y1_body,
        grid=(NB,),
        in_specs=[
            pl.BlockSpec((NC, BN, 16), lambda i: (0, i, 0)),
            pl.BlockSpec((BN, D), lambda i: (i, 0)),
            pl.BlockSpec((D, HID), lambda i: (0, 0)),
        ],
        out_specs=pl.BlockSpec((BN, HID), lambda i: (i, 0)),
        out_shape=jax.ShapeDtypeStruct((N, HID), jnp.float32),
    )(cnt_s, x, w1)


def _gnorm_h(x, s1_ref, s2_ref, al_ref, ga_ref, be_ref):
    mu = s1_ref[...] * (1.0 / N)
    ex2 = s2_ref[...] * (1.0 / N)
    al = al_ref[...]
    var = ex2 - (2.0 * al - al * al) * mu * mu
    sub = x - al * mu
    return _leaky(ga_ref[...] * sub * lax.rsqrt(var + _EPS) + be_ref[...])


def _layer1_body(p_ref, cs_ref, cd_ref, al_ref, ga_ref, be_ref,
                 pw_ref, pb_ref, w2_ref, y2_ref, ps_ref, s1_ref, s2_ref):
    ph = pl.program_id(0)
    i = pl.program_id(1)
    x = (p_ref[0] + p_ref[1]) * _dinv(cd_ref[...])

    @pl.when(ph == 0)
    def _():
        @pl.when(i == 0)
        def _():
            s1_ref[...] = jnp.zeros_like(s1_ref)
            s2_ref[...] = jnp.zeros_like(s2_ref)
        s1_ref[...] += jnp.sum(x, axis=0, keepdims=True)
        s2_ref[...] += jnp.sum(x * x, axis=0, keepdims=True)

    @pl.when(ph == 1)
    def _():
        h = _gnorm_h(x, s1_ref, s2_ref, al_ref, ga_ref, be_ref)
        phis = _leaky(_DOT(h, pw_ref[...]) + pb_ref[...])

        @pl.when(i == 0)
        def _():
            ps_ref[...] = jnp.zeros_like(ps_ref)
        ps_ref[...] += jnp.sum(phis, axis=0, keepdims=True)
        y2_ref[...] = _DOT(h * _dinv(cs_ref[...]), w2_ref[...])


def _layer1_call(p, cnt_s, cnt_d, al, ga, be, pw, pb, w2):
    return pl.pallas_call(
        _layer1_body,
        grid=(2, NB),
        in_specs=[
            pl.BlockSpec((NC, BN, HID), lambda p_, i: (0, i, 0)),
            pl.BlockSpec((NC, BN, 16), lambda p_, i: (0, i, 0)),
            pl.BlockSpec((NC, BN, 16), lambda p_, i: (0, i, 0)),
            pl.BlockSpec((1, HID), lambda p_, i: (0, 0)),
            pl.BlockSpec((1, HID), lambda p_, i: (0, 0)),
            pl.BlockSpec((1, HID), lambda p_, i: (0, 0)),
            pl.BlockSpec((HID, R_HID), lambda p_, i: (0, 0)),
            pl.BlockSpec((1, R_HID), lambda p_, i: (0, 0)),
            pl.BlockSpec((HID, HID), lambda p_, i: (0, 0)),
        ],
        out_specs=[
            pl.BlockSpec((BN, HID), lambda p_, i: (i, 0)),
            pl.BlockSpec((1, R_HID), lambda p_, i: (0, 0)),
        ],
        out_shape=[
            jax.ShapeDtypeStruct((N, HID), jnp.float32),
            jax.ShapeDtypeStruct((1, R_HID), jnp.float32),
        ],
        scratch_shapes=[
            pltpu.VMEM((1, HID), jnp.float32),
            pltpu.VMEM((1, HID), jnp.float32),
        ],
    )(p, cnt_s, cnt_d, al, ga, be, pw, pb, w2)


def _layer2_body(p_ref, cd_ref, al_ref, ga_ref, be_ref, pw_ref, pb_ref,
                 ps1_ref, rw1_ref, rb1_ref, rw2_ref, rb2_ref,
                 o_ref, s1_ref, s2_ref, ps_ref):
    ph = pl.program_id(0)
    i = pl.program_id(1)
    x = (p_ref[0] + p_ref[1]) * _dinv(cd_ref[...])

    @pl.when(ph == 0)
    def _():
        @pl.when(i == 0)
        def _():
            s1_ref[...] = jnp.zeros_like(s1_ref)
            s2_ref[...] = jnp.zeros_like(s2_ref)
        s1_ref[...] += jnp.sum(x, axis=0, keepdims=True)
        s2_ref[...] += jnp.sum(x * x, axis=0, keepdims=True)

    @pl.when(ph == 1)
    def _():
        h = _gnorm_h(x, s1_ref, s2_ref, al_ref, ga_ref, be_ref)
        phis = _leaky(_DOT(h, pw_ref[...]) + pb_ref[...])

        @pl.when(i == 0)
        def _():
            ps_ref[...] = jnp.zeros_like(ps_ref)
        ps_ref[...] += jnp.sum(phis, axis=0, keepdims=True)

        @pl.when(i == NB - 1)
        def _():
            r1 = _leaky(_DOT(ps1_ref[...], rw1_ref[...]) + rb1_ref[...])
            r2 = _leaky(_DOT(ps_ref[...], rw2_ref[...]) + rb2_ref[...])
            o_ref[...] = _leaky(jnp.concatenate([r1, r2], axis=1))


def _layer2_call(p, cnt_d, al, ga, be, pw, pb, ps1, rw1, rb1, rw2, rb2):
    return pl.pallas_call(
        _layer2_body,
        grid=(2, NB),
        in_specs=[
            pl.BlockSpec((NC, BN, HID), lambda p_, i: (0, i, 0)),
            pl.BlockSpec((NC, BN, 16), lambda p_, i: (0, i, 0)),
            pl.BlockSpec((1, HID), lambda p_, i: (0, 0)),
            pl.BlockSpec((1, HID), lambda p_, i: (0, 0)),
            pl.BlockSpec((1, HID), lambda p_, i: (0, 0)),
            pl.BlockSpec((HID, R_HID), lambda p_, i: (0, 0)),
            pl.BlockSpec((1, R_HID), lambda p_, i: (0, 0)),
            pl.BlockSpec((1, R_HID), lambda p_, i: (0, 0)),
            pl.BlockSpec((R_HID, R_OUT), lambda p_, i: (0, 0)),
            pl.BlockSpec((1, R_OUT), lambda p_, i: (0, 0)),
            pl.BlockSpec((R_HID, R_OUT), lambda p_, i: (0, 0)),
            pl.BlockSpec((1, R_OUT), lambda p_, i: (0, 0)),
        ],
        out_specs=pl.BlockSpec((1, 2 * R_OUT), lambda p_, i: (0, 0)),
        out_shape=jax.ShapeDtypeStruct((1, 2 * R_OUT), jnp.float32),
        scratch_shapes=[
            pltpu.VMEM((1, HID), jnp.float32),
            pltpu.VMEM((1, HID), jnp.float32),
            pltpu.VMEM((1, R_HID), jnp.float32),
        ],
    )(p, cnt_d, al, ga, be, pw, pb, ps1, rw1, rb1, rw2, rb2)


def kernel(features, edge_index, edge_weights, W1, W2,
           gn1_alpha, gn1_gamma, gn1_beta, gn2_alpha, gn2_gamma, gn2_beta,
           r1_phi_W, r1_phi_b, r1_rho_W, r1_rho_b,
           r2_phi_W, r2_phi_b, r2_rho_W, r2_rho_b):
    src3 = edge_index[0].reshape(NW, NCHUNK, CK)
    dst3 = edge_index[1].reshape(NW, NCHUNK, CK)

    cnt_s, cnt_d = _deg_call(src3, dst3)

    row = lambda v: v.reshape(1, -1)

    y1 = _y1_call(cnt_s, features, W1)
    p1 = _agg_call(y1, src3, dst3, edge_weights)
    y2, ps1 = _layer1_call(p1, cnt_s, cnt_d,
                           row(gn1_alpha), row(gn1_gamma), row(gn1_beta),
                           r1_phi_W, row(r1_phi_b), W2)
    p2 = _agg_call(y2, src3, dst3, edge_weights)
    return _layer2_call(p2, cnt_d,
                        row(gn2_alpha), row(gn2_gamma), row(gn2_beta),
                        r2_phi_W, row(r2_phi_b),
                        ps1, r1_rho_W, row(r1_rho_b),
                        r2_rho_W, row(r2_rho_b))


# speedup vs baseline: 1.0420x; 1.0048x over previous
import functools

import jax
import jax.numpy as jnp
from jax import lax
from jax.experimental import pallas as pl
from jax.experimental.pallas import tpu as pltpu
from jax.experimental.pallas import tpu_sc as plsc

N = 10000
E = 320000
D = 128
HID = 128
R_HID = 256
R_OUT = 32

NC = 2
NS = 16
NW = NC * NS
EPW = E // NW
CK = 80
NCHUNK = EPW // CK
TF = 640
TL = 400
WBF = 128
WBL = 80
NWB = 5

_EPS = 1e-5


def _leaky(x):
    return jnp.where(x >= 0, x, 0.01 * x)


def _sc_mesh():
    return plsc.VectorSubcoreMesh(
        core_axis_name="c", subcore_axis_name="s", num_cores=NC, num_subcores=NS
    )


_SC_PARAMS = pltpu.CompilerParams(use_tc_tiling_on_sc=False,
                                  needs_layout_passes=False)


def _deg_body(src_hbm, dst_hbm, cs_hbm, cd_hbm,
              idx_s, idx_d, ones_v, tmp16, sem, acc_s, acc_d):
    c = lax.axis_index("c")
    s = lax.axis_index("s")
    wid = c * NS + s
    row0 = s * TF
    last = s == NS - 1

    pltpu.sync_copy(src_hbm.at[wid], idx_s)
    pltpu.sync_copy(dst_hbm.at[wid], idx_d)

    def _fill(i, _):
        ones_v[i, :] = jnp.ones((16,), jnp.float32)
        return 0
    lax.fori_loop(0, CK, _fill, 0)

    def _zero(i, _):
        tmp16[i, :] = jnp.zeros((16,), jnp.float32)
        return 0
    lax.fori_loop(0, TF, _zero, 0)

    @pl.when(jnp.logical_not(last))
    def _():
        pltpu.sync_copy(tmp16, acc_s.at[pl.ds(row0, TF)])
        pltpu.sync_copy(tmp16, acc_d.at[pl.ds(row0, TF)])

    @pl.when(last)
    def _():
        pltpu.sync_copy(tmp16.at[pl.ds(0, TL)], acc_s.at[pl.ds(row0, TL)])
        pltpu.sync_copy(tmp16.at[pl.ds(0, TL)], acc_d.at[pl.ds(row0, TL)])

    plsc.subcore_barrier()

    def _chunk(g, _):
        h1 = pltpu.async_copy(ones_v, acc_s.at[idx_s.at[g]], sem, add=True)
        h2 = pltpu.async_copy(ones_v, acc_d.at[idx_d.at[g]], sem, add=True)
        h1.wait()
        h2.wait()
        return 0
    lax.fori_loop(0, NCHUNK, _chunk, 0)
    plsc.subcore_barrier()

    def _write(acc, out_hbm, nrows):
        pltpu.sync_copy(acc.at[pl.ds(row0, nrows)],
                        out_hbm.at[c, pl.ds(row0, nrows)])

    @pl.when(jnp.logical_not(last))
    def _():
        _write(acc_s, cs_hbm, TF)
        _write(acc_d, cd_hbm, TF)

    @pl.when(last)
    def _():
        _write(acc_s, cs_hbm, TL)
        _write(acc_d, cd_hbm, TL)


def _deg_call(src3, dst3):
    return pl.kernel(
        _deg_body,
        out_type=(
            jax.ShapeDtypeStruct((NC, N, 16), jnp.float32),
            jax.ShapeDtypeStruct((NC, N, 16), jnp.float32),
        ),
        mesh=_sc_mesh(),
        compiler_params=_SC_PARAMS,
        scratch_types=(
            pltpu.VMEM((NCHUNK, CK), jnp.int32),
            pltpu.VMEM((NCHUNK, CK), jnp.int32),
            pltpu.VMEM((CK, 16), jnp.float32),
            pltpu.VMEM((TF, 16), jnp.float32),
            pltpu.SemaphoreType.DMA,
            pltpu.VMEM_SHARED((N, 16), jnp.float32),
            pltpu.VMEM_SHARED((N, 16), jnp.float32),
        ),
    )(src3, dst3)


NSEG = 5
QS = NCHUNK // NSEG
ESEG = EPW // NSEG
ZWB = 80


def _agg_body(y_hbm, src_hbm, dst_hbm, ew_hbm, p_hbm,
              sidx, didx, ewh, rows0, rows1, rows2, rows3,
              gs0, gs1, gs2, gs3, ss0, ss1, ss2, ss3, acc):
    c = lax.axis_index("c")
    s = lax.axis_index("s")
    wid = c * NS + s
    row0 = s * TF
    last = s == NS - 1
    nzwb = TF // ZWB

    def _zrow(i, _):
        for j in range(8):
            rows0[i, pl.ds(j * 16, 16)] = jnp.zeros((16,), jnp.float32)
        return 0
    lax.fori_loop(0, ZWB, _zrow, 0)

    @pl.when(jnp.logical_not(last))
    def _():
        for k in range(nzwb):
            pltpu.sync_copy(rows0.at[pl.ds(0, ZWB)],
                            acc.at[pl.ds(row0 + k * ZWB, ZWB)])

    @pl.when(last)
    def _():
        for k in range(TL // ZWB):
            pltpu.sync_copy(rows0.at[pl.ds(0, ZWB)],
                            acc.at[pl.ds(row0 + k * ZWB, ZWB)])

    plsc.subcore_barrier()

    def _scale(rows, g):
        def _one(t, _):
            k0 = 4 * t
            ws = [plsc.load_gather(
                      ewh, [jnp.full((16,), g * CK + k0 + u, jnp.int32)])
                  for u in range(4)]
            for u in range(4):
                for j in range(8):
                    rows[k0 + u, pl.ds(j * 16, 16)] = (
                        rows[k0 + u, pl.ds(j * 16, 16)] * ws[u])
            return 0
        lax.fori_loop(0, CK // 4, _one, 0)

    bufs = ((rows0, gs0, ss0), (rows1, gs1, ss1),
            (rows2, gs2, ss2), (rows3, gs3, ss3))
    NBUF = len(bufs)
    PF = NBUF - 1

    for q in range(NSEG):
        pltpu.sync_copy(src_hbm.at[wid, pl.ds(q * QS, QS)], sidx)
        pltpu.sync_copy(dst_hbm.at[wid, pl.ds(q * QS, QS)], didx)
        pltpu.sync_copy(ew_hbm.at[pl.ds(wid * EPW + q * ESEG, ESEG)], ewh)

        for g0 in range(PF):
            pltpu.async_copy(y_hbm.at[sidx.at[g0]], bufs[g0][0], bufs[g0][1])

        def _chunk(g, _):
            b = lax.rem(g, NBUF)
            for i, (buf, gsem, ssem) in enumerate(bufs):
                @pl.when(b == i)
                def _(buf=buf, gsem=gsem, ssem=ssem):
                    pltpu.make_async_copy(
                        y_hbm.at[sidx.at[g]], buf, gsem).wait()
                    _scale(buf, g)
                    pltpu.async_copy(buf, acc.at[didx.at[g]], ssem, add=True)

            gn = g + PF
            bn = lax.rem(gn, NBUF)

            @pl.when(gn < QS)
            def _():
                for i, (buf, gsem, ssem) in enumerate(bufs):
                    @pl.when(bn == i)
                    def _(buf=buf, gsem=gsem, ssem=ssem):
                        @pl.when(g >= 1)
                        def _():
                            pltpu.make_async_copy(
                                buf, acc.at[didx.at[g - 1]], ssem).wait()
                        pltpu.async_copy(y_hbm.at[sidx.at[gn]], buf, gsem)

            return 0
        lax.fori_loop(0, QS, _chunk, 0)

        for g in range(QS - NBUF, QS):
            buf, _, ssem = bufs[g % NBUF]
            pltpu.make_async_copy(buf, acc.at[didx.at[g]], ssem).wait()
    plsc.subcore_barrier()

    @pl.when(jnp.logical_not(last))
    def _():
        for k in range(nzwb):
            pltpu.sync_copy(acc.at[pl.ds(row0 + k * ZWB, ZWB)],
                            p_hbm.at[c, pl.ds(row0 + k * ZWB, ZWB)])

    @pl.when(last)
    def _():
        for k in range(TL // ZWB):
            pltpu.sync_copy(acc.at[pl.ds(row0 + k * ZWB, ZWB)],
                            p_hbm.at[c, pl.ds(row0 + k * ZWB, ZWB)])


def _agg_call(y, src3, dst3, ew):
    return pl.kernel(
        _agg_body,
        out_type=jax.ShapeDtypeStruct((NC, N, D), jnp.float32),
        mesh=_sc_mesh(),
        compiler_params=_SC_PARAMS,
        scratch_types=(
            pltpu.VMEM((QS, CK), jnp.int32),
            pltpu.VMEM((QS, CK), jnp.int32),
            pltpu.VMEM((ESEG,), jnp.float32),
            pltpu.VMEM((CK, D), jnp.float32),
            pltpu.VMEM((CK, D), jnp.float32),
            pltpu.VMEM((CK, D), jnp.float32),
            pltpu.VMEM((CK, D), jnp.float32),
            pltpu.SemaphoreType.DMA,
            pltpu.SemaphoreType.DMA,
            pltpu.SemaphoreType.DMA,
            pltpu.SemaphoreType.DMA,
            pltpu.SemaphoreType.DMA,
            pltpu.SemaphoreType.DMA,
            pltpu.SemaphoreType.DMA,
            pltpu.SemaphoreType.DMA,
            pltpu.VMEM_SHARED((N, D), jnp.float32),
        ),
    )(y, src3, dst3, ew)


BN = 1000
NB = N // BN

_DOT = functools.partial(jnp.dot, preferred_element_type=jnp.float32,
                         precision=lax.Precision.HIGHEST)


def _dinv(cnt_block):
    deg = jnp.maximum(cnt_block[0, :, 0:1] + cnt_block[1, :, 0:1], 1.0)
    return lax.rsqrt(deg)


def _y1_body(cs_ref, x_ref, w_ref, y_ref):
    h = x_ref[...] * _dinv(cs_ref[...])
    y_ref[...] = _DOT(h, w_ref[...])


def _y1_call(cnt_s, x, w1):
    return pl.pallas_call(
        _y1_body,
        grid=(NB,),
        in_specs=[
            pl.BlockSpec((NC, BN, 16), lambda i: (0, i, 0)),
            pl.BlockSpec((BN, D), lambda i: (i, 0)),
            pl.BlockSpec((D, HID), lambda i: (0, 0)),
        ],
        out_specs=pl.BlockSpec((BN, HID), lambda i: (i, 0)),
        out_shape=jax.ShapeDtypeStruct((N, HID), jnp.float32),
    )(cnt_s, x, w1)


def _gnorm_h(x, s1_ref, s2_ref, al_ref, ga_ref, be_ref):
    mu = s1_ref[...] * (1.0 / N)
    ex2 = s2_ref[...] * (1.0 / N)
    al = al_ref[...]
    var = ex2 - (2.0 * al - al * al) * mu * mu
    sub = x - al * mu
    return _leaky(ga_ref[...] * sub * lax.rsqrt(var + _EPS) + be_ref[...])


def _layer1_body(p_ref, cs_ref, cd_ref, al_ref, ga_ref, be_ref,
                 pw_ref, pb_ref, w2_ref, y2_ref, ps_ref, s1_ref, s2_ref):
    ph = pl.program_id(0)
    i = pl.program_id(1)
    x = (p_ref[0] + p_ref[1]) * _dinv(cd_ref[...])

    @pl.when(ph == 0)
    def _():
        @pl.when(i == 0)
        def _():
            s1_ref[...] = jnp.zeros_like(s1_ref)
            s2_ref[...] = jnp.zeros_like(s2_ref)
        s1_ref[...] += jnp.sum(x, axis=0, keepdims=True)
        s2_ref[...] += jnp.sum(x * x, axis=0, keepdims=True)

    @pl.when(ph == 1)
    def _():
        h = _gnorm_h(x, s1_ref, s2_ref, al_ref, ga_ref, be_ref)
        phis = _leaky(_DOT(h, pw_ref[...]) + pb_ref[...])

        @pl.when(i == 0)
        def _():
            ps_ref[...] = jnp.zeros_like(ps_ref)
        ps_ref[...] += jnp.sum(phis, axis=0, keepdims=True)
        y2_ref[...] = _DOT(h * _dinv(cs_ref[...]), w2_ref[...])


def _layer1_call(p, cnt_s, cnt_d, al, ga, be, pw, pb, w2):
    return pl.pallas_call(
        _layer1_body,
        grid=(2, NB),
        in_specs=[
            pl.BlockSpec((NC, BN, HID), lambda p_, i: (0, i, 0)),
            pl.BlockSpec((NC, BN, 16), lambda p_, i: (0, i, 0)),
            pl.BlockSpec((NC, BN, 16), lambda p_, i: (0, i, 0)),
            pl.BlockSpec((1, HID), lambda p_, i: (0, 0)),
            pl.BlockSpec((1, HID), lambda p_, i: (0, 0)),
            pl.BlockSpec((1, HID), lambda p_, i: (0, 0)),
            pl.BlockSpec((HID, R_HID), lambda p_, i: (0, 0)),
            pl.BlockSpec((1, R_HID), lambda p_, i: (0, 0)),
            pl.BlockSpec((HID, HID), lambda p_, i: (0, 0)),
        ],
        out_specs=[
            pl.BlockSpec((BN, HID), lambda p_, i: (i, 0)),
            pl.BlockSpec((1, R_HID), lambda p_, i: (0, 0)),
        ],
        out_shape=[
            jax.ShapeDtypeStruct((N, HID), jnp.float32),
            jax.ShapeDtypeStruct((1, R_HID), jnp.float32),
        ],
        scratch_shapes=[
            pltpu.VMEM((1, HID), jnp.float32),
            pltpu.VMEM((1, HID), jnp.float32),
        ],
    )(p, cnt_s, cnt_d, al, ga, be, pw, pb, w2)


def _layer2_body(p_ref, cd_ref, al_ref, ga_ref, be_ref, pw_ref, pb_ref,
                 ps1_ref, rw1_ref, rb1_ref, rw2_ref, rb2_ref,
                 o_ref, s1_ref, s2_ref, ps_ref):
    ph = pl.program_id(0)
    i = pl.program_id(1)
    x = (p_ref[0] + p_ref[1]) * _dinv(cd_ref[...])

    @pl.when(ph == 0)
    def _():
        @pl.when(i == 0)
        def _():
            s1_ref[...] = jnp.zeros_like(s1_ref)
            s2_ref[...] = jnp.zeros_like(s2_ref)
        s1_ref[...] += jnp.sum(x, axis=0, keepdims=True)
        s2_ref[...] += jnp.sum(x * x, axis=0, keepdims=True)

    @pl.when(ph == 1)
    def _():
        h = _gnorm_h(x, s1_ref, s2_ref, al_ref, ga_ref, be_ref)
        phis = _leaky(_DOT(h, pw_ref[...]) + pb_ref[...])

        @pl.when(i == 0)
        def _():
            ps_ref[...] = jnp.zeros_like(ps_ref)
        ps_ref[...] += jnp.sum(phis, axis=0, keepdims=True)

        @pl.when(i == NB - 1)
        def _():
            r1 = _leaky(_DOT(ps1_ref[...], rw1_ref[...]) + rb1_ref[...])
            r2 = _leaky(_DOT(ps_ref[...], rw2_ref[...]) + rb2_ref[...])
            o_ref[...] = _leaky(jnp.concatenate([r1, r2], axis=1))


def _layer2_call(p, cnt_d, al, ga, be, pw, pb, ps1, rw1, rb1, rw2, rb2):
    return pl.pallas_call(
        _layer2_body,
        grid=(2, NB),
        in_specs=[
            pl.BlockSpec((NC, BN, HID), lambda p_, i: (0, i, 0)),
            pl.BlockSpec((NC, BN, 16), lambda p_, i: (0, i, 0)),
            pl.BlockSpec((1, HID), lambda p_, i: (0, 0)),
            pl.BlockSpec((1, HID), lambda p_, i: (0, 0)),
            pl.BlockSpec((1, HID), lambda p_, i: (0, 0)),
            pl.BlockSpec((HID, R_HID), lambda p_, i: (0, 0)),
            pl.BlockSpec((1, R_HID), lambda p_, i: (0, 0)),
            pl.BlockSpec((1, R_HID), lambda p_, i: (0, 0)),
            pl.BlockSpec((R_HID, R_OUT), lambda p_, i: (0, 0)),
            pl.BlockSpec((1, R_OUT), lambda p_, i: (0, 0)),
            pl.BlockSpec((R_HID, R_OUT), lambda p_, i: (0, 0)),
            pl.BlockSpec((1, R_OUT), lambda p_, i: (0, 0)),
        ],
        out_specs=pl.BlockSpec((1, 2 * R_OUT), lambda p_, i: (0, 0)),
        out_shape=jax.ShapeDtypeStruct((1, 2 * R_OUT), jnp.float32),
        scratch_shapes=[
            pltpu.VMEM((1, HID), jnp.float32),
            pltpu.VMEM((1, HID), jnp.float32),
            pltpu.VMEM((1, R_HID), jnp.float32),
        ],
    )(p, cnt_d, al, ga, be, pw, pb, ps1, rw1, rb1, rw2, rb2)


def kernel(features, edge_index, edge_weights, W1, W2,
           gn1_alpha, gn1_gamma, gn1_beta, gn2_alpha, gn2_gamma, gn2_beta,
           r1_phi_W, r1_phi_b, r1_rho_W, r1_rho_b,
           r2_phi_W, r2_phi_b, r2_rho_W, r2_rho_b):
    src3 = edge_index[0].reshape(NW, NCHUNK, CK)
    dst3 = edge_index[1].reshape(NW, NCHUNK, CK)

    cnt_s, cnt_d = _deg_call(src3, dst3)

    row = lambda v: v.reshape(1, -1)

    y1 = _y1_call(cnt_s, features, W1)
    p1 = _agg_call(y1, src3, dst3, edge_weights)
    y2, ps1 = _layer1_call(p1, cnt_s, cnt_d,
                           row(gn1_alpha), row(gn1_gamma), row(gn1_beta),
                           r1_phi_W, row(r1_phi_b), W2)
    p2 = _agg_call(y2, src3, dst3, edge_weights)
    return _layer2_call(p2, cnt_d,
                        row(gn2_alpha), row(gn2_gamma), row(gn2_beta),
                        r2_phi_W, row(r2_phi_b),
                        ps1, r1_rho_W, row(r1_rho_b),
                        r2_rho_W, row(r2_rho_b))
